# baseline (speedup 1.0000x reference)
import functools

import jax
import jax.numpy as jnp
from jax import lax
from jax.experimental import pallas as pl
from jax.experimental.pallas import tpu as pltpu
from jax.experimental.pallas import tpu_sc as plsc

CLS_ID = 1
B, S, D = 4, 8192, 2048
NS, L = 16, 16
TILES_PER_ROW = NS // B
CHUNK = S // TILES_PER_ROW
UNROLL = 8
ITERS = CHUNK // (L * UNROLL)


def _sc_pooler(table, ids):
    mesh = plsc.VectorSubcoreMesh(core_axis_name="c", subcore_axis_name="s",
                                  num_cores=1)

    @functools.partial(
        pl.kernel,
        out_type=jax.ShapeDtypeStruct((B, D), jnp.float32),
        mesh=mesh,
        scratch_types=[
            pltpu.VMEM_SHARED((NS, L), jnp.int32),
            pltpu.VMEM((CHUNK,), jnp.int32),
            pltpu.VMEM((L,), jnp.int32),
            pltpu.VMEM((TILES_PER_ROW, L), jnp.int32),
            pltpu.VMEM((L,), jnp.int32),
            pltpu.VMEM((1, D), jnp.float32),
            pltpu.SemaphoreType.DMA,
        ],
    )
    def body(table_hbm, ids_hbm, out_hbm,
             shared, ids_v, acc_v, mine_v, idx_v, row_v, sem):
        sid = lax.axis_index("s")
        row = sid // TILES_PER_ROW
        chunk = sid % TILES_PER_ROW
        base = row * S + chunk * CHUNK
        pltpu.sync_copy(ids_hbm.at[pl.ds(base, CHUNK)], ids_v)

        lanes = lax.broadcasted_iota(jnp.int32, (L,), 0)
        pos0 = lanes + chunk * CHUNK
        neg = jnp.full((L,), -1, jnp.int32)

        @plsc.parallel_loop(0, CHUNK, L, unroll=UNROLL, carry=neg)
        def acc(i, acc):
            v = ids_v[pl.ds(i, L)]
            pos = pos0 + i
            return jnp.maximum(acc, jnp.where(v == CLS_ID, pos, neg))
        for sh in (1, 2, 4, 8):
            acc = jnp.maximum(acc, acc.at[lanes ^ sh].get(
                mode="promise_in_bounds"))
        acc_v[...] = acc
        pltpu.sync_copy(acc_v, shared.at[sid])
        plsc.subcore_barrier()

        @pl.when(sid < B)
        def _reduce_and_gather():
            pltpu.sync_copy(shared.at[pl.ds(sid * TILES_PER_ROW,
                                            TILES_PER_ROW)], mine_v)
            m = mine_v[0]
            for t in range(1, TILES_PER_ROW):
                m = jnp.maximum(m, mine_v[t])
            m = jnp.where(m < 0, m + S, m)
            idx_v[...] = sid * S + m
            pltpu.async_copy(table_hbm.at[idx_v.at[pl.ds(0, 1)]],
                             row_v, sem).wait()
            pltpu.sync_copy(row_v, out_hbm.at[pl.ds(sid, 1)])

    return body(table, ids)


def kernel(layer_states, input_ids, return_final):
    del return_final
    ids = input_ids.astype(jnp.int32).reshape(-1)
    table = layer_states.reshape(B * S, D)
    return _sc_pooler(table, ids)


# speedup vs baseline: 1.0156x; 1.0156x over previous
import functools

import jax
import jax.numpy as jnp
from jax import lax
from jax.experimental import pallas as pl
from jax.experimental.pallas import tpu as pltpu
from jax.experimental.pallas import tpu_sc as plsc

CLS_ID = 1
B, S, D = 4, 8192, 2048
NS, L = 16, 16
TILES_PER_ROW = NS // B
CHUNK = S // TILES_PER_ROW
UNROLL = 16
ITERS = CHUNK // (L * UNROLL)


def _sc_pooler(table, ids):
    mesh = plsc.VectorSubcoreMesh(core_axis_name="c", subcore_axis_name="s",
                                  num_cores=1)

    @functools.partial(
        pl.kernel,
        out_type=jax.ShapeDtypeStruct((B, D), jnp.float32),
        mesh=mesh,
        scratch_types=[
            pltpu.VMEM_SHARED((NS, L), jnp.int32),
            pltpu.VMEM((CHUNK,), jnp.int32),
            pltpu.VMEM((L,), jnp.int32),
            pltpu.VMEM((TILES_PER_ROW, L), jnp.int32),
            pltpu.VMEM((L,), jnp.int32),
            pltpu.VMEM((1, D), jnp.float32),
            pltpu.SemaphoreType.DMA,
        ],
    )
    def body(table_hbm, ids_hbm, out_hbm,
             shared, ids_v, acc_v, mine_v, idx_v, row_v, sem):
        sid = lax.axis_index("s")
        row = sid // TILES_PER_ROW
        chunk = sid % TILES_PER_ROW
        base = row * S + chunk * CHUNK
        pltpu.sync_copy(ids_hbm.at[pl.ds(base, CHUNK)], ids_v)

        lanes = lax.broadcasted_iota(jnp.int32, (L,), 0)
        pos0 = lanes + chunk * CHUNK
        neg = jnp.full((L,), -1, jnp.int32)

        def step(i, acc):
            for j in range(UNROLL):
                v = ids_v[pl.ds((i * UNROLL + j) * L, L)]
                pos = pos0 + (i * UNROLL + j) * L
                acc = jnp.maximum(acc, jnp.where(v == CLS_ID, pos, neg))
            return acc

        acc = lax.fori_loop(0, ITERS, step, neg)
        for sh in (1, 2, 4, 8):
            acc = jnp.maximum(acc, acc.at[lanes ^ sh].get(
                mode="promise_in_bounds"))
        acc_v[...] = acc
        pltpu.sync_copy(acc_v, shared.at[sid])
        plsc.subcore_barrier()

        @pl.when(sid < B)
        def _reduce_and_gather():
            pltpu.sync_copy(shared.at[pl.ds(sid * TILES_PER_ROW,
                                            TILES_PER_ROW)], mine_v)
            m = mine_v[0]
            for t in range(1, TILES_PER_ROW):
                m = jnp.maximum(m, mine_v[t])
            m = jnp.where(m < 0, m + S, m)
            idx_v[...] = sid * S + m
            pltpu.async_copy(table_hbm.at[idx_v.at[pl.ds(0, 1)]],
                             row_v, sem).wait()
            pltpu.sync_copy(row_v, out_hbm.at[pl.ds(sid, 1)])

    return body(table, ids)


def kernel(layer_states, input_ids, return_final):
    del return_final
    ids = input_ids.astype(jnp.int32).reshape(-1)
    table = layer_states.reshape(B * S, D)
    return _sc_pooler(table, ids)
